# baseline (device time: 112514 ns/iter reference)
import jax
import jax.numpy as jnp
from jax import lax
from jax.experimental import pallas as pl
from jax.experimental.pallas import tpu as pltpu

P = 8

FLAT_SLOTS = (
    (1, 0, 0), (0, 1, 1),
    (0, 1, 0), (1, 0, 1),
    (0, 0, 1), (1, 1, 0),
    (1, 1, 1),
)


def _coords_of_pos(p):
    q = p % 4
    x = jnp.where((q == 1) | (q == 2), 1, 0)
    y = jnp.where(q >= 2, 1, 0)
    return x, y, p // 4


def _pos_of_coords(x, y, z):
    return 2 * y + jnp.where(y == 0, x, 1 - x) + 4 * z


def kernel(x):
    m, n_global = x.shape
    n = n_global // P
    m_out = P * m
    nf = P - 1

    def body(x_ref, out_ref, xf32_ref, xbf_ref, own_f32_ref, own_bf_ref,
             stage_sems, send_sems, recv_sems, out_sem):
        my = lax.axis_index("i")
        cx, cy, cz = _coords_of_pos(my)
        peers = [
            _pos_of_coords(cx ^ dx, cy ^ dy, cz ^ dz)
            for dx, dy, dz in FLAT_SLOTS
        ]

        stages = []
        for f in range(nf):
            st = pltpu.make_async_copy(
                x_ref.at[:, pl.ds(peers[f] * n, n)],
                xf32_ref.at[:, pl.ds(f * n, n)],
                stage_sems.at[f],
            )
            st.start()
            stages.append(st)
        own_stage = pltpu.make_async_copy(
            x_ref.at[:, pl.ds(my * n, n)], own_f32_ref, stage_sems.at[nf]
        )
        own_stage.start()

        barrier = pltpu.get_barrier_semaphore()
        for d in range(1, P):
            peer = lax.rem(my + d, P)
            pl.semaphore_signal(
                barrier, inc=1,
                device_id=(peer,), device_id_type=pl.DeviceIdType.MESH,
            )
        pl.semaphore_wait(barrier, P - 1)

        rdmas = []
        for f in range(nf):
            stages[f].wait()
            xbf_ref[:, pl.ds(f * n, n)] = (
                xf32_ref[:, pl.ds(f * n, n)].astype(jnp.bfloat16)
            )
            rdma = pltpu.make_async_remote_copy(
                src_ref=xbf_ref.at[:, pl.ds(f * n, n)],
                dst_ref=out_ref.at[pl.ds(my * m, m), :],
                send_sem=send_sems.at[f],
                recv_sem=recv_sems.at[f],
                device_id=(peers[f],),
                device_id_type=pl.DeviceIdType.MESH,
            )
            rdma.start()
            rdmas.append(rdma)

        own_stage.wait()
        own_bf_ref[...] = own_f32_ref[...].astype(jnp.bfloat16)
        own_copy = pltpu.make_async_copy(
            own_bf_ref, out_ref.at[pl.ds(my * m, m), :], out_sem
        )
        own_copy.start()
        own_copy.wait()

        for rdma in rdmas:
            rdma.wait_send()
        for rdma in rdmas:
            rdma.wait_recv()

    return pl.pallas_call(
        body,
        out_shape=jax.ShapeDtypeStruct((m_out, n), jnp.bfloat16),
        in_specs=[pl.BlockSpec(memory_space=pltpu.MemorySpace.HBM)],
        out_specs=pl.BlockSpec(memory_space=pltpu.MemorySpace.HBM),
        scratch_shapes=[
            pltpu.VMEM((m, nf * n), jnp.float32),
            pltpu.VMEM((m, nf * n), jnp.bfloat16),
            pltpu.VMEM((m, n), jnp.float32),
            pltpu.VMEM((m, n), jnp.bfloat16),
            pltpu.SemaphoreType.DMA((P,)),
            pltpu.SemaphoreType.DMA((nf,)),
            pltpu.SemaphoreType.DMA((nf,)),
            pltpu.SemaphoreType.DMA,
        ],
        compiler_params=pltpu.CompilerParams(
            collective_id=0, vmem_limit_bytes=64 * 1024 * 1024
        ),
    )(x)


# device time: 112085 ns/iter; 1.0038x vs baseline; 1.0038x over previous
import jax
import jax.numpy as jnp
from jax import lax
from jax.experimental import pallas as pl
from jax.experimental.pallas import tpu as pltpu

P = 8

FLAT_SLOTS = (
    (1, 1, 1),
    (1, 0, 0), (0, 1, 1),
    (0, 1, 0), (1, 0, 1),
    (0, 0, 1), (1, 1, 0),
)


def _coords_of_pos(p):
    q = p % 4
    x = jnp.where((q == 1) | (q == 2), 1, 0)
    y = jnp.where(q >= 2, 1, 0)
    return x, y, p // 4


def _pos_of_coords(x, y, z):
    return 2 * y + jnp.where(y == 0, x, 1 - x) + 4 * z


def kernel(x):
    m, n_global = x.shape
    n = n_global // P
    m_out = P * m
    nf = P - 1

    def body(x_ref, out_ref, xf32_ref, xbf_ref, own_f32_ref, own_bf_ref,
             stage_sems, send_sems, recv_sems, out_sem):
        my = lax.axis_index("i")
        cx, cy, cz = _coords_of_pos(my)
        peers = [
            _pos_of_coords(cx ^ dx, cy ^ dy, cz ^ dz)
            for dx, dy, dz in FLAT_SLOTS
        ]

        stages = []
        for f in range(nf):
            st = pltpu.make_async_copy(
                x_ref.at[:, pl.ds(peers[f] * n, n)],
                xf32_ref.at[:, pl.ds(f * n, n)],
                stage_sems.at[f],
            )
            st.start()
            stages.append(st)
        own_stage = pltpu.make_async_copy(
            x_ref.at[:, pl.ds(my * n, n)], own_f32_ref, stage_sems.at[nf]
        )
        own_stage.start()

        barrier = pltpu.get_barrier_semaphore()
        for d in range(1, P):
            peer = lax.rem(my + d, P)
            pl.semaphore_signal(
                barrier, inc=1,
                device_id=(peer,), device_id_type=pl.DeviceIdType.MESH,
            )
        pl.semaphore_wait(barrier, P - 1)

        rdmas = []
        for f in range(nf):
            stages[f].wait()
            xbf_ref[:, pl.ds(f * n, n)] = (
                xf32_ref[:, pl.ds(f * n, n)].astype(jnp.bfloat16)
            )
            rdma = pltpu.make_async_remote_copy(
                src_ref=xbf_ref.at[:, pl.ds(f * n, n)],
                dst_ref=out_ref.at[pl.ds(my * m, m), :],
                send_sem=send_sems.at[f],
                recv_sem=recv_sems.at[f],
                device_id=(peers[f],),
                device_id_type=pl.DeviceIdType.MESH,
            )
            rdma.start()
            rdmas.append(rdma)

        own_stage.wait()
        own_bf_ref[...] = own_f32_ref[...].astype(jnp.bfloat16)
        own_copy = pltpu.make_async_copy(
            own_bf_ref, out_ref.at[pl.ds(my * m, m), :], out_sem
        )
        own_copy.start()

        for rdma in rdmas:
            rdma.wait_send()
        own_copy.wait()
        for rdma in rdmas:
            rdma.wait_recv()

    return pl.pallas_call(
        body,
        out_shape=jax.ShapeDtypeStruct((m_out, n), jnp.bfloat16),
        in_specs=[pl.BlockSpec(memory_space=pltpu.MemorySpace.HBM)],
        out_specs=pl.BlockSpec(memory_space=pltpu.MemorySpace.HBM),
        scratch_shapes=[
            pltpu.VMEM((m, nf * n), jnp.float32),
            pltpu.VMEM((m, nf * n), jnp.bfloat16),
            pltpu.VMEM((m, n), jnp.float32),
            pltpu.VMEM((m, n), jnp.bfloat16),
            pltpu.SemaphoreType.DMA((P,)),
            pltpu.SemaphoreType.DMA((nf,)),
            pltpu.SemaphoreType.DMA((nf,)),
            pltpu.SemaphoreType.DMA,
        ],
        compiler_params=pltpu.CompilerParams(
            collective_id=0, vmem_limit_bytes=64 * 1024 * 1024
        ),
    )(x)
